# baseline (speedup 1.0000x reference)
import functools
import math

import jax
import jax.numpy as jnp
from jax import lax
from jax.experimental import pallas as pl
from jax.experimental.pallas import tpu as pltpu
from jax.experimental.pallas import tpu_sc as plsc

D = 64
NC, NS = 2, 16
NW = NC * NS
CH = 128
SCALE = math.sqrt(D)


@functools.partial(jax.jit, static_argnums=(2,))
def _emb_lookup(idx2d, lut, n_ch_w):
    n_total = idx2d.shape[0] * CH
    mesh = plsc.VectorSubcoreMesh(
        core_axis_name="c", subcore_axis_name="s", num_cores=NC, num_subcores=NS
    )

    @functools.partial(
        pl.kernel,
        out_type=jax.ShapeDtypeStruct((n_total, D), jnp.float32),
        mesh=mesh,
        scratch_types=[
            pltpu.VMEM((n_ch_w, CH), jnp.int32),
            pltpu.VMEM((CH, D), jnp.float32),
            pltpu.SemaphoreType.DMA,
        ],
        compiler_params=pltpu.CompilerParams(use_tc_tiling_on_sc=False),
    )
    def k(idx_hbm, table_hbm, out_hbm, idx_v, rows_v, sem):
        wid = lax.axis_index("s") * NC + lax.axis_index("c")
        chunk0 = wid * n_ch_w
        pltpu.sync_copy(idx_hbm.at[pl.ds(chunk0, n_ch_w)], idx_v)

        def chunk_body(j, _):
            pltpu.async_copy(table_hbm.at[idx_v.at[j]], rows_v, sem).wait()

            @plsc.parallel_loop(0, CH, unroll=4)
            def _(r):
                for c in range(D // 16):
                    sl = pl.ds(c * 16, 16)
                    rows_v[r, sl] = rows_v[r, sl] * SCALE

            base = (chunk0 + j) * CH
            pltpu.sync_copy(rows_v, out_hbm.at[pl.ds(base, CH)])
            return 0

        lax.fori_loop(0, n_ch_w, chunk_body, 0)

    return k(idx2d, lut)


def kernel(x, lut):
    B = x.size
    assert B % (NW * CH) == 0
    n_ch_w = B // (NW * CH)
    idx2d = x.reshape(B // CH, CH).astype(jnp.int32)
    out = _emb_lookup(idx2d, lut, n_ch_w)
    return out.reshape(*x.shape, D)
